# baseline (speedup 1.0000x reference)
import functools

import jax
import jax.numpy as jnp
from jax import lax
from jax.experimental import pallas as pl
from jax.experimental.pallas import tpu as pltpu
from jax.experimental.pallas import tpu_sc as plsc


def _sc_features(coords, t0, t1, h0, h1):
    tokens = coords.shape[1]
    info = plsc.get_sparse_core_info()
    ncores, nsub = info.num_cores, info.num_subcores
    nw = ncores * nsub
    tpw = tokens // nw
    assert tpw * nw == tokens
    ch = 640
    assert tpw % ch == 0
    n_chunks = tpw // ch
    n_groups = ch // 16

    n0, d0 = t0.shape
    n1, d1 = t1.shape
    nh0, dh = h0.shape
    nh1, _ = h1.shape
    nfeat = 2 * (d0 + d1) + 2 * dh

    mesh = plsc.VectorSubcoreMesh(core_axis_name="c", subcore_axis_name="s")

    @functools.partial(
        pl.kernel,
        out_type=jax.ShapeDtypeStruct((nfeat, tokens), jnp.float32),
        mesh=mesh,
        compiler_params=pltpu.CompilerParams(needs_layout_passes=False),
        scratch_types=[
            pltpu.VMEM((n0 * d0,), jnp.float32),
            pltpu.VMEM((n1 * d1,), jnp.float32),
            pltpu.VMEM((nh0 * dh,), jnp.float32),
            pltpu.VMEM((nh1 * dh,), jnp.float32),
            pltpu.VMEM((3, ch), jnp.float32),
            pltpu.VMEM((nfeat, ch), jnp.float32),
        ],
    )
    def sc_kernel(co_h, t0_h, t1_h, h0_h, h1_h, out_h,
                  t0_v, t1_v, h0_v, h1_v, co_v, xb_v):
        wid = lax.axis_index("s") * ncores + lax.axis_index("c")
        base = wid * tpw
        pltpu.sync_copy(t0_h, t0_v)
        pltpu.sync_copy(t1_h, t1_v)
        pltpu.sync_copy(h0_h, h0_v)
        pltpu.sync_copy(h1_h, h1_v)

        def chunk_body(c, carry):
            tok0 = base + c * ch
            pltpu.sync_copy(co_h.at[:, pl.ds(tok0, ch)], co_v)

            def group_body(g, carry2):
                s = g * 16
                px = co_v[0, pl.ds(s, 16)]
                py = co_v[1, pl.ds(s, 16)]
                hd = co_v[2, pl.ds(s, 16)]
                ex = px + 300.0
                ix0 = jnp.clip(ex.astype(jnp.int32), 0, n0 - 1)
                rx = ex - ix0.astype(jnp.float32)
                ix1 = jnp.clip((rx / 0.01).astype(jnp.int32), 0, n1 - 1)
                ey = py + 300.0
                iy0 = jnp.clip(ey.astype(jnp.int32), 0, n0 - 1)
                ry = ey - iy0.astype(jnp.float32)
                iy1 = jnp.clip((ry / 0.01).astype(jnp.int32), 0, n1 - 1)
                eh = hd * 180.0 / jnp.pi + 180.0
                ih0 = jnp.clip((eh / 20.0).astype(jnp.int32), 0, nh0 - 1)
                rh = eh - ih0.astype(jnp.float32) * 20.0
                ih1 = jnp.clip(rh.astype(jnp.int32), 0, nh1 - 1)

                taps = [
                    (t0_v, ix0 * d0, d0, 0),
                    (t1_v, ix1 * d1, d1, d0),
                    (t0_v, iy0 * d0, d0, d0 + d1),
                    (t1_v, iy1 * d1, d1, 2 * d0 + d1),
                    (h0_v, ih0 * dh, dh, 2 * (d0 + d1)),
                    (h1_v, ih1 * dh, dh, 2 * (d0 + d1) + dh),
                ]
                items = [(tab, gidx, j, row0 + j)
                         for tab, gidx, d, row0 in taps for j in range(d)]
                for w0 in range(0, len(items), 16):
                    wave = items[w0:w0 + 16]
                    vals = [plsc.load_gather(tab, [gidx + j])
                            for tab, gidx, j, _ in wave]
                    for (_, _, _, row), v in zip(wave, vals):
                        xb_v[row, pl.ds(s, 16)] = v
                return carry2

            lax.fori_loop(0, n_groups, group_body, 0)
            pltpu.sync_copy(xb_v, out_h.at[:, pl.ds(tok0, ch)])
            return carry

        lax.fori_loop(0, n_chunks, chunk_body, 0)

    return sc_kernel(coords, t0.reshape(-1), t1.reshape(-1),
                     h0.reshape(-1), h1.reshape(-1))


def _ln_relu_bf16(x, g, b, eps=1e-5):
    bf16 = jnp.bfloat16
    var = jnp.mean(x * x, axis=-1, keepdims=True)
    r = lax.rsqrt(var + eps)
    y = ((x * r).astype(bf16)) * g + b
    return jnp.maximum(y, jnp.zeros((), bf16))


def _tc_mlp(xt, maskf, w1, g1, b1, w2, g2, b2, w3, b3, oob):
    nfeat, tokens = xt.shape
    tile = 4096
    grid = tokens // tile
    assert grid * tile == tokens

    def body(x_ref, m_ref, w1_ref, g1_ref, b1_ref, w2_ref, g2_ref, b2_ref,
             w3_ref, b3_ref, oob_ref, o_ref):
        bf16 = jnp.bfloat16
        x = x_ref[...].astype(bf16)
        h = lax.dot_general(x, w1_ref[...], (((0,), (0,)), ((), ())),
                            preferred_element_type=jnp.float32)
        h = _ln_relu_bf16(h, g1_ref[...], b1_ref[...])
        h = jnp.dot(h, w2_ref[...], preferred_element_type=jnp.float32)
        h = _ln_relu_bf16(h, g2_ref[...], b2_ref[...])
        y = jnp.dot(h, w3_ref[...], preferred_element_type=jnp.float32)
        y = y + b3_ref[...]
        m = m_ref[...]
        o_ref[...] = jnp.where(m > 0.0, y, oob_ref[...])

    full = lambda shape: pl.BlockSpec(shape, lambda i: (0, 0))
    return pl.pallas_call(
        body,
        grid=(grid,),
        in_specs=[
            pl.BlockSpec((nfeat, tile), lambda i: (0, i)),
            pl.BlockSpec((tile, 1), lambda i: (i, 0)),
            full((nfeat, 256)),
            full((1, 256)),
            full((1, 256)),
            full((256, 256)),
            full((1, 256)),
            full((1, 256)),
            full((256, 256)),
            full((1, 256)),
            full((1, 256)),
        ],
        out_specs=pl.BlockSpec((tile, 256), lambda i: (i, 0)),
        out_shape=jax.ShapeDtypeStruct((tokens, 256), jnp.float32),
    )(xt, maskf, w1, g1, b1, w2, g2, b2, w3, b3, oob)


def kernel(agent_position, agent_heading, agent_valid_mask, map_polygon_center,
           map_valid_mask, pos_table_0, pos_table_1, head_table_0, head_table_1,
           w1, ln1_g, ln1_b, w2, ln2_g, ln2_b, w3, b3, oob_w, window_T):
    B, N, T = agent_heading.shape
    tokens = B * (T - 1) * N

    posx = jnp.swapaxes(agent_position[:, :, 1:, 0], 1, 2).reshape(-1)
    posy = jnp.swapaxes(agent_position[:, :, 1:, 1], 1, 2).reshape(-1)
    hd = jnp.swapaxes(agent_heading[:, :, 1:], 1, 2).reshape(-1)
    coords = jnp.stack([posx, posy, hd], axis=0)
    maskf = jnp.swapaxes(agent_valid_mask[:, :, 1:], 1, 2).reshape(-1, 1)
    maskf = maskf.astype(jnp.float32)

    xt = _sc_features(coords, pos_table_0, pos_table_1,
                      head_table_0, head_table_1)

    bf16 = jnp.bfloat16
    w1 = w1 - jnp.mean(w1, axis=1, keepdims=True)
    w2 = w2 - jnp.mean(w2, axis=1, keepdims=True)
    out = _tc_mlp(xt, maskf, w1.astype(bf16),
                  ln1_g.reshape(1, -1).astype(bf16),
                  ln1_b.reshape(1, -1).astype(bf16),
                  w2.astype(bf16),
                  ln2_g.reshape(1, -1).astype(bf16),
                  ln2_b.reshape(1, -1).astype(bf16),
                  w3.astype(bf16), b3.reshape(1, -1), oob_w)
    return out.reshape(B, T - 1, N, 256)


# speedup vs baseline: 1.3281x; 1.3281x over previous
import functools

import jax
import jax.numpy as jnp
from jax import lax
from jax.experimental import pallas as pl
from jax.experimental.pallas import tpu as pltpu
from jax.experimental.pallas import tpu_sc as plsc


def _sc_features(coords, t0, t1, h0, h1):
    tokens = coords.shape[1]
    info = plsc.get_sparse_core_info()
    ncores, nsub = info.num_cores, info.num_subcores
    nw = ncores * nsub
    tpw = tokens // nw
    assert tpw * nw == tokens
    ch = 384
    assert tpw % ch == 0
    n_chunks = tpw // ch
    n_groups = ch // 16

    n0, d0 = t0.shape
    n1, d1 = t1.shape
    nh0, dh = h0.shape
    nh1, _ = h1.shape
    nfeat = 2 * (d0 + d1) + 2 * dh

    mesh = plsc.VectorSubcoreMesh(core_axis_name="c", subcore_axis_name="s")

    @functools.partial(
        pl.kernel,
        out_type=jax.ShapeDtypeStruct((nfeat, tokens), jnp.float32),
        mesh=mesh,
        compiler_params=pltpu.CompilerParams(needs_layout_passes=False),
        scratch_types=[
            pltpu.VMEM((n0 * d0,), jnp.float32),
            pltpu.VMEM((n1 * d1,), jnp.float32),
            pltpu.VMEM((nh0 * dh,), jnp.float32),
            pltpu.VMEM((nh1 * dh,), jnp.float32),
            pltpu.VMEM((3, ch), jnp.float32),
            pltpu.VMEM((3, ch), jnp.float32),
            pltpu.VMEM((nfeat, ch), jnp.float32),
            pltpu.VMEM((nfeat, ch), jnp.float32),
            pltpu.SemaphoreType.DMA,
            pltpu.SemaphoreType.DMA,
            pltpu.SemaphoreType.DMA,
            pltpu.SemaphoreType.DMA,
        ],
    )
    def sc_kernel(co_h, t0_h, t1_h, h0_h, h1_h, out_h,
                  t0_v, t1_v, h0_v, h1_v, co0_v, co1_v, xb0_v, xb1_v,
                  so0, so1, si0, si1):
        wid = lax.axis_index("s") * ncores + lax.axis_index("c")
        base = wid * tpw
        pltpu.sync_copy(t0_h, t0_v)
        pltpu.sync_copy(t1_h, t1_v)
        pltpu.sync_copy(h0_h, h0_v)
        pltpu.sync_copy(h1_h, h1_v)
        co_bufs = (co0_v, co1_v)
        xb_bufs = (xb0_v, xb1_v)
        sout = (so0, so1)
        sin = (si0, si1)

        def compute_chunk(co_v, xb_v):

            def group_body(g, carry2):
                s = g * 16
                px = co_v[0, pl.ds(s, 16)]
                py = co_v[1, pl.ds(s, 16)]
                hd = co_v[2, pl.ds(s, 16)]
                ex = px + 300.0
                ix0 = jnp.clip(ex.astype(jnp.int32), 0, n0 - 1)
                rx = ex - ix0.astype(jnp.float32)
                ix1 = jnp.clip((rx / 0.01).astype(jnp.int32), 0, n1 - 1)
                ey = py + 300.0
                iy0 = jnp.clip(ey.astype(jnp.int32), 0, n0 - 1)
                ry = ey - iy0.astype(jnp.float32)
                iy1 = jnp.clip((ry / 0.01).astype(jnp.int32), 0, n1 - 1)
                eh = hd * 180.0 / jnp.pi + 180.0
                ih0 = jnp.clip((eh / 20.0).astype(jnp.int32), 0, nh0 - 1)
                rh = eh - ih0.astype(jnp.float32) * 20.0
                ih1 = jnp.clip(rh.astype(jnp.int32), 0, nh1 - 1)

                taps = [
                    (t0_v, ix0 * d0, d0, 0),
                    (t1_v, ix1 * d1, d1, d0),
                    (t0_v, iy0 * d0, d0, d0 + d1),
                    (t1_v, iy1 * d1, d1, 2 * d0 + d1),
                    (h0_v, ih0 * dh, dh, 2 * (d0 + d1)),
                    (h1_v, ih1 * dh, dh, 2 * (d0 + d1) + dh),
                ]
                items = [(tab, gidx, j, row0 + j)
                         for tab, gidx, d, row0 in taps for j in range(d)]
                for w0 in range(0, len(items), 16):
                    wave = items[w0:w0 + 16]
                    vals = [plsc.load_gather(tab, [gidx + j])
                            for tab, gidx, j, _ in wave]
                    for (_, _, _, row), v in zip(wave, vals):
                        xb_v[row, pl.ds(s, 16)] = v
                return carry2

            lax.fori_loop(0, n_groups, group_body, 0)

        def in_copy(c, b):
            tok0 = base + c * ch
            return pltpu.make_async_copy(
                co_h.at[:, pl.ds(tok0, ch)], co_bufs[b], sin[b])

        def out_copy(c, b):
            tok0 = base + c * ch
            return pltpu.make_async_copy(
                xb_bufs[b], out_h.at[:, pl.ds(tok0, ch)], sout[b])

        in_copy(0, 0).start()
        in_copy(1, 1).start()
        in_copy(0, 0).wait()
        compute_chunk(co_bufs[0], xb_bufs[0])
        out_copy(0, 0).start()

        n_pairs = (n_chunks - 1) // 2

        def pair_body(c2, carry):
            c_a = 1 + 2 * c2
            c_b = 2 + 2 * c2

            in_copy(c_b, 0).start()
            in_copy(c_a, 1).wait()

            @pl.when(c2 > 0)
            def _():
                out_copy(c_a - 2, 1).wait()

            compute_chunk(co_bufs[1], xb_bufs[1])
            out_copy(c_a, 1).start()

            @pl.when(c2 + 1 < n_pairs)
            def _():
                in_copy(c_a + 2, 1).start()

            in_copy(c_b, 0).wait()
            out_copy(c_b - 2, 0).wait()
            compute_chunk(co_bufs[0], xb_bufs[0])
            out_copy(c_b, 0).start()
            return carry

        lax.fori_loop(0, n_pairs, pair_body, 0)
        out_copy(n_chunks - 2, 1).wait()
        out_copy(n_chunks - 1, 0).wait()

    return sc_kernel(coords, t0.reshape(-1), t1.reshape(-1),
                     h0.reshape(-1), h1.reshape(-1))


def _ln_relu_bf16(x, g, b, eps=1e-5):
    bf16 = jnp.bfloat16
    mu = jnp.mean(x, axis=-1, keepdims=True)
    mu2 = jnp.mean(x * x, axis=-1, keepdims=True)
    var = jnp.maximum(mu2 - mu * mu, 0.0)
    r = lax.rsqrt(var + eps)
    y = ((x - mu) * r).astype(bf16) * g + b
    return jnp.maximum(y, jnp.zeros((), bf16))


def _tc_mlp(xt, maskf, w1, g1, b1, w2, g2, b2, w3, b3, oob):
    nfeat, tokens = xt.shape
    tile = 4096
    grid = tokens // tile
    assert grid * tile == tokens

    def body(x_ref, m_ref, w1_ref, g1_ref, b1_ref, w2_ref, g2_ref, b2_ref,
             w3_ref, b3_ref, oob_ref, o_ref):
        bf16 = jnp.bfloat16
        x = x_ref[...].astype(bf16)
        h = lax.dot_general(x, w1_ref[...], (((0,), (0,)), ((), ())),
                            preferred_element_type=jnp.float32)
        h = _ln_relu_bf16(h, g1_ref[...], b1_ref[...])
        h = jnp.dot(h, w2_ref[...], preferred_element_type=jnp.float32)
        h = _ln_relu_bf16(h, g2_ref[...], b2_ref[...])
        y = jnp.dot(h, w3_ref[...], preferred_element_type=jnp.float32)
        y = y + b3_ref[...]
        m = m_ref[...]
        o_ref[...] = jnp.where(m > 0.0, y, oob_ref[...])

    full = lambda shape: pl.BlockSpec(shape, lambda i: (0, 0))
    return pl.pallas_call(
        body,
        grid=(grid,),
        in_specs=[
            pl.BlockSpec((nfeat, tile), lambda i: (0, i)),
            pl.BlockSpec((tile, 1), lambda i: (i, 0)),
            full((nfeat, 256)),
            full((1, 256)),
            full((1, 256)),
            full((256, 256)),
            full((1, 256)),
            full((1, 256)),
            full((256, 256)),
            full((1, 256)),
            full((1, 256)),
        ],
        out_specs=pl.BlockSpec((tile, 256), lambda i: (i, 0)),
        out_shape=jax.ShapeDtypeStruct((tokens, 256), jnp.float32),
    )(xt, maskf, w1, g1, b1, w2, g2, b2, w3, b3, oob)


def kernel(agent_position, agent_heading, agent_valid_mask, map_polygon_center,
           map_valid_mask, pos_table_0, pos_table_1, head_table_0, head_table_1,
           w1, ln1_g, ln1_b, w2, ln2_g, ln2_b, w3, b3, oob_w, window_T):
    B, N, T = agent_heading.shape
    tokens = B * (T - 1) * N

    posx = jnp.swapaxes(agent_position[:, :, 1:, 0], 1, 2).reshape(-1)
    posy = jnp.swapaxes(agent_position[:, :, 1:, 1], 1, 2).reshape(-1)
    hd = jnp.swapaxes(agent_heading[:, :, 1:], 1, 2).reshape(-1)
    coords = jnp.stack([posx, posy, hd], axis=0)
    maskf = jnp.swapaxes(agent_valid_mask[:, :, 1:], 1, 2).reshape(-1, 1)
    maskf = maskf.astype(jnp.float32)

    xt = _sc_features(coords, pos_table_0, pos_table_1,
                      head_table_0, head_table_1)

    bf16 = jnp.bfloat16
    out = _tc_mlp(xt, maskf, w1.astype(bf16),
                  ln1_g.reshape(1, -1).astype(bf16),
                  ln1_b.reshape(1, -1).astype(bf16),
                  w2.astype(bf16),
                  ln2_g.reshape(1, -1).astype(bf16),
                  ln2_b.reshape(1, -1).astype(bf16),
                  w3.astype(bf16), b3.reshape(1, -1), oob_w)
    return out.reshape(B, T - 1, N, 256)
